# baseline (speedup 1.0000x reference)
import jax
import jax.numpy as jnp
from jax import lax
from jax.experimental import pallas as pl
from jax.experimental.pallas import tpu as pltpu
from jax.experimental.pallas import tpu_sc as plsc

B, S, H = 4, 2048, 768
V, P, T = 30522, 2048, 2
EPS = 1e-12

NC, NS, L = 2, 16, 16
NW = NC * NS
SPW = S // NW
NJ = H // L
INV_H = 1.0 / H


def _body(ids_hbm, tt_hbm, word_hbm, pos_hbm, type_hbm, gamma_hbm, beta_hbm,
          out_hbm, idx_v, ttv_v, pos_v, row_v, type_v, g_v, b_v, sem):
    cid = lax.axis_index("c")
    sid = lax.axis_index("s")
    wid = sid * NC + cid
    s0 = pl.multiple_of(wid * SPW, SPW)

    pltpu.sync_copy(pos_hbm.at[pl.ds(s0, SPW)], pos_v)
    pltpu.sync_copy(type_hbm, type_v)
    pltpu.sync_copy(gamma_hbm, g_v)
    pltpu.sync_copy(beta_hbm, b_v)

    for j in range(NJ):
        sl = pl.ds(j * L, L)
        type_v[1, sl] = type_v[1, sl] - type_v[0, sl]

    def fold_type0(i, c):
        for j in range(NJ):
            sl = pl.ds(j * L, L)
            pos_v[i, sl] = pos_v[i, sl] + type_v[0, sl]
        return c
    lax.fori_loop(0, SPW, fold_type0, 0)

    def per_batch(b, c):
        tok0 = pl.multiple_of(b * S + s0, SPW)
        pltpu.sync_copy(ids_hbm.at[pl.ds(tok0, SPW)], idx_v)
        pltpu.sync_copy(tt_hbm.at[pl.ds(tok0, SPW)], ttv_v)
        cp = pltpu.make_async_copy(word_hbm.at[idx_v], row_v, sem)
        cp.start()
        cp.wait()

        def per_token(i, c2):
            lane_i = jnp.broadcast_to(i, (L,))
            ttf = plsc.load_gather(ttv_v, [lane_i]).astype(jnp.float32)
            acc = jnp.zeros((L,), jnp.float32)
            acc2 = jnp.zeros((L,), jnp.float32)
            for j in range(NJ):
                sl = pl.ds(j * L, L)
                e = row_v[i, sl] + pos_v[i, sl] + ttf * type_v[1, sl]
                row_v[i, sl] = e
                acc = acc + e
                acc2 = acc2 + e * e
            mean = jnp.sum(acc) * INV_H
            var = jnp.sum(acc2) * INV_H - mean * mean
            xv = jnp.broadcast_to(var + EPS, (L,))
            yi = plsc.bitcast(xv, jnp.int32)
            y = plsc.bitcast(jnp.int32(0x5F3759DF) - (yi >> 1), jnp.float32)
            for _ in range(4):
                y = y * (1.5 - 0.5 * xv * y * y)
            meanv = jnp.broadcast_to(mean, (L,))
            for j in range(NJ):
                sl = pl.ds(j * L, L)
                row_v[i, sl] = (row_v[i, sl] - meanv) * y * g_v[sl] + b_v[sl]
            return c2
        lax.fori_loop(0, SPW, per_token, 0)

        pltpu.sync_copy(row_v, out_hbm.at[pl.ds(tok0, SPW)])
        return c
    lax.fori_loop(0, B, per_batch, 0)


@jax.jit
def _run(ids, tt, word_table, pos_table, type_table, gamma, beta):
    mesh = plsc.VectorSubcoreMesh(core_axis_name="c", subcore_axis_name="s",
                                  num_cores=NC, num_subcores=NS)
    return pl.kernel(
        _body,
        out_type=jax.ShapeDtypeStruct((B * S, H), jnp.float32),
        mesh=mesh,
        compiler_params=pltpu.CompilerParams(needs_layout_passes=False),
        scratch_types=[
            pltpu.VMEM((SPW,), jnp.int32),
            pltpu.VMEM((SPW,), jnp.int32),
            pltpu.VMEM((SPW, H), jnp.float32),
            pltpu.VMEM((SPW, H), jnp.float32),
            pltpu.VMEM((T, H), jnp.float32),
            pltpu.VMEM((H,), jnp.float32),
            pltpu.VMEM((H,), jnp.float32),
            pltpu.SemaphoreType.DMA,
        ],
    )(ids, tt, word_table, pos_table, type_table, gamma, beta)


def kernel(input_ids, token_type_ids, word_table, pos_table, type_table,
           gamma, beta):
    ids = input_ids.reshape(-1).astype(jnp.int32)
    tt = token_type_ids.reshape(-1).astype(jnp.int32)
    out = _run(ids, tt, word_table, pos_table, type_table, gamma, beta)
    return out.reshape(B, S, H)


# speedup vs baseline: 1.0182x; 1.0182x over previous
import jax
import jax.numpy as jnp
from jax import lax
from jax.experimental import pallas as pl
from jax.experimental.pallas import tpu as pltpu
from jax.experimental.pallas import tpu_sc as plsc

B, S, H = 4, 2048, 768
V, P, T = 30522, 2048, 2
EPS = 1e-12

NC, NS, L = 2, 16, 16
NW = NC * NS
SPW = S // NW
TPW = B * SPW
CHUNK = 16
NCH = TPW // CHUNK
NHB = SPW // CHUNK
NJ = H // L
INV_H = 1.0 / H


def _body(ids_hbm, tt_hbm, word_hbm, pos_hbm, type_hbm, gamma_hbm, beta_hbm,
          out_hbm, ids_v, tts_v, pos_v, rowA, rowB, outA, outB, type_v,
          g_v, b_v, sgA, sgB, soA, soB):
    cid = lax.axis_index("c")
    sid = lax.axis_index("s")
    wid = sid * NC + cid
    s0 = pl.multiple_of(wid * SPW, SPW)

    pltpu.sync_copy(pos_hbm.at[pl.ds(s0, SPW)], pos_v)
    pltpu.sync_copy(type_hbm, type_v)
    pltpu.sync_copy(gamma_hbm, g_v)
    pltpu.sync_copy(beta_hbm, b_v)
    for b in range(B):
        tok0 = pl.multiple_of(b * S + s0, SPW)
        lo = b * SPW
        pltpu.sync_copy(ids_hbm.at[pl.ds(tok0, SPW)], ids_v.at[pl.ds(lo, SPW)])
        pltpu.sync_copy(tt_hbm.at[pl.ds(tok0, SPW)], tts_v.at[pl.ds(lo, SPW)])

    for j in range(NJ):
        sl = pl.ds(j * L, L)
        type_v[1, sl] = type_v[1, sl] - type_v[0, sl]

    def fold_type0(i, c):
        for j in range(NJ):
            sl = pl.ds(j * L, L)
            pos_v[i, sl] = pos_v[i, sl] + type_v[0, sl]
        return c
    lax.fori_loop(0, SPW, fold_type0, 0)

    def gather_cp(k, row_ref, sem):
        idxv = ids_v[pl.ds(k * CHUNK, CHUNK)]
        return pltpu.make_async_copy(word_hbm.at[idxv], row_ref, sem)

    def out_cp(k, out_ref, sem):
        b = k // NHB
        h = k % NHB
        tok0 = pl.multiple_of(b * S + s0 + h * CHUNK, CHUNK)
        return pltpu.make_async_copy(out_ref, out_hbm.at[pl.ds(tok0, CHUNK)],
                                     sem)

    def compute(k, row_ref, out_ref):
        hbase = (k % NHB) * CHUNK

        def per_token(i, c2):
            lane_i = jnp.broadcast_to(k * CHUNK + i, (L,))
            ttf = plsc.load_gather(tts_v, [lane_i]).astype(jnp.float32)
            ip = hbase + i
            acc = jnp.zeros((L,), jnp.float32)
            acc2 = jnp.zeros((L,), jnp.float32)
            for j in range(NJ):
                sl = pl.ds(j * L, L)
                e = row_ref[i, sl] + pos_v[ip, sl] + ttf * type_v[1, sl]
                out_ref[i, sl] = e
                acc = acc + e
                acc2 = acc2 + e * e
            mean = jnp.sum(acc) * INV_H
            var = jnp.sum(acc2) * INV_H - mean * mean
            xv = jnp.broadcast_to(var + EPS, (L,))
            yi = plsc.bitcast(xv, jnp.int32)
            y = plsc.bitcast(jnp.int32(0x5F3759DF) - (yi >> 1), jnp.float32)
            for _ in range(4):
                y = y * (1.5 - 0.5 * xv * y * y)
            meanv = jnp.broadcast_to(mean, (L,))
            for j in range(NJ):
                sl = pl.ds(j * L, L)
                out_ref[i, sl] = (out_ref[i, sl] - meanv) * y * g_v[sl] \
                    + b_v[sl]
            return c2
        lax.fori_loop(0, CHUNK, per_token, 0)

    gather_cp(0, rowA, sgA).start()

    def pair(p, c):
        kA = 2 * p
        kB = kA + 1
        gather_cp(kA, rowA, sgA).wait()
        gather_cp(kB, rowB, sgB).start()

        @pl.when(p >= 1)
        def _():
            out_cp(kA - 2, outA, soA).wait()
        compute(kA, rowA, outA)
        out_cp(kA, outA, soA).start()

        gather_cp(kB, rowB, sgB).wait()

        @pl.when(p <= (NCH // 2) - 2)
        def _():
            gather_cp(kA + 2, rowA, sgA).start()

        @pl.when(p >= 1)
        def _():
            out_cp(kB - 2, outB, soB).wait()
        compute(kB, rowB, outB)
        out_cp(kB, outB, soB).start()
        return c
    lax.fori_loop(0, NCH // 2, pair, 0)

    out_cp(NCH - 2, outA, soA).wait()
    out_cp(NCH - 1, outB, soB).wait()


@jax.jit
def _run(ids, tt, word_table, pos_table, type_table, gamma, beta):
    mesh = plsc.VectorSubcoreMesh(core_axis_name="c", subcore_axis_name="s",
                                  num_cores=NC, num_subcores=NS)
    return pl.kernel(
        _body,
        out_type=jax.ShapeDtypeStruct((B * S, H), jnp.float32),
        mesh=mesh,
        compiler_params=pltpu.CompilerParams(needs_layout_passes=False),
        scratch_types=[
            pltpu.VMEM((TPW,), jnp.int32),
            pltpu.VMEM((TPW,), jnp.int32),
            pltpu.VMEM((SPW, H), jnp.float32),
            pltpu.VMEM((CHUNK, H), jnp.float32),
            pltpu.VMEM((CHUNK, H), jnp.float32),
            pltpu.VMEM((CHUNK, H), jnp.float32),
            pltpu.VMEM((CHUNK, H), jnp.float32),
            pltpu.VMEM((T, H), jnp.float32),
            pltpu.VMEM((H,), jnp.float32),
            pltpu.VMEM((H,), jnp.float32),
            pltpu.SemaphoreType.DMA,
            pltpu.SemaphoreType.DMA,
            pltpu.SemaphoreType.DMA,
            pltpu.SemaphoreType.DMA,
        ],
    )(ids, tt, word_table, pos_table, type_table, gamma, beta)


def kernel(input_ids, token_type_ids, word_table, pos_table, type_table,
           gamma, beta):
    ids = input_ids.reshape(-1).astype(jnp.int32)
    tt = token_type_ids.reshape(-1).astype(jnp.int32)
    out = _run(ids, tt, word_table, pos_table, type_table, gamma, beta)
    return out.reshape(B, S, H)
